# baseline (speedup 1.0000x reference)
import functools

import jax
import jax.numpy as jnp
from jax import lax
from jax.experimental import pallas as pl
from jax.experimental.pallas import tpu as pltpu
from jax.experimental.pallas import tpu_sc as plsc

N = 10000
NP = 10240
E = 320000
D = 128
NC = 2
NS = 16
NW = NC * NS
EPW = E // NW
CH = 80
NCH = EPW // CH
SEC = 5
SCH = NCH // SEC
RPT = NP // NS


def _make_sc(with_cnt: bool):
    mesh = plsc.VectorSubcoreMesh(core_axis_name="c", subcore_axis_name="s")
    out_type = [jax.ShapeDtypeStruct((NC, NP, D), jnp.float32)]
    scratch = [
        pltpu.VMEM((2, SCH, CH), jnp.int32),
        pltpu.VMEM((2, SCH, CH), jnp.int32),
        pltpu.VMEM((CH, D), jnp.float32),
        pltpu.VMEM((CH, D), jnp.float32),
        pltpu.VMEM_SHARED((NP, D), jnp.float32),
        pltpu.SemaphoreType.DMA,
        pltpu.SemaphoreType.DMA,
        pltpu.SemaphoreType.DMA,
        pltpu.SemaphoreType.DMA,
        pltpu.SemaphoreType.DMA,
        pltpu.VMEM((RPT,), jnp.float32),
    ]
    if with_cnt:
        out_type.append(jax.ShapeDtypeStruct((NC, NP), jnp.float32))
        scratch += [
            pltpu.VMEM((CH,), jnp.float32),
            pltpu.VMEM_SHARED((NP,), jnp.float32),
        ]

    @functools.partial(pl.kernel, mesh=mesh, out_type=out_type,
                       scratch_types=scratch)
    def sc(x_hbm, ei_hbm, *refs):
        if with_cnt:
            (sums_hbm, cnt_hbm, src_sec, dst_sec, rows_a, rows_b, acc_sh,
             sem_a, sem_b, isem_0, isem_1, csem, zero_v,
             ones_v, cnt_sh) = refs
        else:
            (sums_hbm, src_sec, dst_sec, rows_a, rows_b, acc_sh,
             sem_a, sem_b, isem_0, isem_1, csem, zero_v) = refs
        isems = (isem_0, isem_1)
        c = lax.axis_index("c")
        s = lax.axis_index("s")
        wid = s * NC + c

        pltpu.async_copy(ei_hbm.at[0, wid, 0], src_sec.at[0], isem_0)
        pltpu.async_copy(ei_hbm.at[1, wid, 0], dst_sec.at[0], isem_0)

        def zfill(i, carry):
            for j in range(D // 16):
                rows_a[i, pl.ds(j * 16, 16)] = jnp.zeros((16,), jnp.float32)
            return carry
        lax.fori_loop(0, CH, zfill, 0)
        for m in range(RPT // CH):
            pltpu.sync_copy(rows_a,
                            acc_sh.at[pl.ds(s * RPT + m * CH, CH)])
        if with_cnt:
            def zfill1(i, carry):
                zero_v[pl.ds(i * 16, 16)] = jnp.zeros((16,), jnp.float32)
                return carry
            lax.fori_loop(0, RPT // 16, zfill1, 0)
            pltpu.sync_copy(zero_v, cnt_sh.at[pl.ds(s * RPT, RPT)])
            for j in range(CH // 16):
                ones_v[pl.ds(j * 16, 16)] = jnp.ones((16,), jnp.float32)
        plsc.subcore_barrier()

        def leg(b, k, rows_v, sem):
            pltpu.make_async_copy(x_hbm.at[src_sec.at[b, 0]], rows_v,
                                  sem).wait()
            pltpu.sync_copy(rows_v, acc_sh.at[dst_sec.at[b, k]], add=True)
            if with_cnt:
                pltpu.async_copy(ones_v, cnt_sh.at[dst_sec.at[b, k]], csem,
                                 add=True)

            @pl.when(k + 2 < SCH)
            def _():
                pltpu.async_copy(x_hbm.at[src_sec.at[b, k + 2]], rows_v, sem)

        for s2 in range(SEC):
            b = s2 % 2
            nb = (s2 + 1) % 2
            if s2 + 1 < SEC:
                pltpu.async_copy(ei_hbm.at[0, wid, s2 + 1], src_sec.at[nb],
                                 isems[nb])
                pltpu.async_copy(ei_hbm.at[1, wid, s2 + 1], dst_sec.at[nb],
                                 isems[nb])
            pltpu.make_async_copy(ei_hbm.at[0, wid, 0], src_sec.at[b],
                                  isems[b]).wait()
            pltpu.make_async_copy(ei_hbm.at[1, wid, 0], dst_sec.at[b],
                                  isems[b]).wait()
            pltpu.async_copy(x_hbm.at[src_sec.at[b, 0]], rows_a, sem_a)
            pltpu.async_copy(x_hbm.at[src_sec.at[b, 1]], rows_b, sem_b)

            def pair(p, carry):
                k0 = 2 * p
                leg(b, k0, rows_a, sem_a)

                @pl.when(k0 + 1 < SCH)
                def _():
                    leg(b, k0 + 1, rows_b, sem_b)
                return carry

            lax.fori_loop(0, (SCH + 1) // 2, pair, 0)
        if with_cnt:
            def drain_cnt(k, carry):
                pltpu.make_async_copy(ones_v, cnt_sh.at[dst_sec.at[0, 0]],
                                      csem).wait()
                return carry
            lax.fori_loop(0, NCH, drain_cnt, 0)
        plsc.subcore_barrier()

        pltpu.sync_copy(acc_sh.at[pl.ds(s * RPT, RPT)],
                        sums_hbm.at[c, pl.ds(s * RPT, RPT)])
        if with_cnt:
            @pl.when(s == 0)
            def _():
                pltpu.sync_copy(cnt_sh, cnt_hbm.at[c])

    return sc


def _make_tc(final: bool):
    R = 2048
    grid = NP // R

    def body(sums_ref, cnt_ref, x_ref, wl_ref, bl_ref, wr_ref, br_ref, o_ref):
        sm = sums_ref[0] + sums_ref[1]
        ct = cnt_ref[0] + cnt_ref[1]
        mean = sm / jnp.clip(ct, 1.0, None)[:, None]
        dn = (((1,), (1,)), ((), ()))
        h = (lax.dot_general(mean, wl_ref[...], dn,
                             precision=lax.Precision.DEFAULT)
             + bl_ref[...]
             + lax.dot_general(x_ref[...], wr_ref[...], dn,
                               precision=lax.Precision.DEFAULT)
             + br_ref[...])
        h = jnp.where(h > 0, h, jnp.exp(jnp.minimum(h, 0.0)) - 1.0)
        if final:
            m = jnp.max(h, axis=-1, keepdims=True)
            sh = h - m
            h = sh - jnp.log(jnp.sum(jnp.exp(sh), axis=-1, keepdims=True))
        o_ref[...] = h

    return pl.pallas_call(
        body,
        grid=(grid,),
        in_specs=[
            pl.BlockSpec((NC, R, D), lambda i: (0, i, 0)),
            pl.BlockSpec((NC, R), lambda i: (0, i)),
            pl.BlockSpec((R, D), lambda i: (i, 0)),
            pl.BlockSpec((D, D), lambda i: (0, 0)),
            pl.BlockSpec((1, D), lambda i: (0, 0)),
            pl.BlockSpec((D, D), lambda i: (0, 0)),
            pl.BlockSpec((1, D), lambda i: (0, 0)),
        ],
        out_specs=pl.BlockSpec((R, D), lambda i: (i, 0)),
        out_shape=jax.ShapeDtypeStruct((N, D), jnp.float32),
    )


def kernel(x, edge_index, W1l, b1l, W1r, b1r, W2l, b2l, W2r, b2r):
    ei = edge_index.reshape(2, NW, SEC, SCH, CH)

    sc1 = _make_sc(with_cnt=True)
    tc1 = _make_tc(final=False)
    tc2 = _make_tc(final=True)

    sums1, cnt = sc1(x, ei)
    h = tc1(sums1, cnt, x, W1l, b1l.reshape(1, D), W1r, b1r.reshape(1, D))
    sums2, _ = sc1(h, ei)
    return tc2(sums2, cnt, h, W2l, b2l.reshape(1, D), W2r, b2r.reshape(1, D))


# speedup vs baseline: 1.0053x; 1.0053x over previous
import functools

import jax
import jax.numpy as jnp
from jax import lax
from jax.experimental import pallas as pl
from jax.experimental.pallas import tpu as pltpu
from jax.experimental.pallas import tpu_sc as plsc

N = 10000
NP = 10240
E = 320000
D = 128
NC = 2
NS = 16
NW = NC * NS
EPW = E // NW
CH = 80
NCH = EPW // CH
SEC = 5
SCH = NCH // SEC
RPT = NP // NS


def _make_sc(with_cnt: bool):
    mesh = plsc.VectorSubcoreMesh(core_axis_name="c", subcore_axis_name="s")
    out_type = [jax.ShapeDtypeStruct((NC, NP, D), jnp.float32)]
    scratch = [
        pltpu.VMEM((2, SCH, CH), jnp.int32),
        pltpu.VMEM((2, SCH, CH), jnp.int32),
        pltpu.VMEM((CH, D), jnp.float32),
        pltpu.VMEM((CH, D), jnp.float32),
        pltpu.VMEM_SHARED((NP, D), jnp.float32),
        pltpu.SemaphoreType.DMA,
        pltpu.SemaphoreType.DMA,
        pltpu.SemaphoreType.DMA,
        pltpu.SemaphoreType.DMA,
        pltpu.SemaphoreType.DMA,
        pltpu.VMEM((RPT,), jnp.float32),
    ]
    if with_cnt:
        out_type.append(jax.ShapeDtypeStruct((NC, NP), jnp.float32))
        scratch += [
            pltpu.VMEM((CH,), jnp.float32),
            pltpu.VMEM_SHARED((NP,), jnp.float32),
        ]

    @functools.partial(pl.kernel, mesh=mesh, out_type=out_type,
                       scratch_types=scratch)
    def sc(x_hbm, ei_hbm, *refs):
        if with_cnt:
            (sums_hbm, cnt_hbm, src_sec, dst_sec, rows_a, rows_b, acc_sh,
             sem_a, sem_b, isem_0, isem_1, csem, zero_v,
             ones_v, cnt_sh) = refs
        else:
            (sums_hbm, src_sec, dst_sec, rows_a, rows_b, acc_sh,
             sem_a, sem_b, isem_0, isem_1, csem, zero_v) = refs
        isems = (isem_0, isem_1)
        c = lax.axis_index("c")
        s = lax.axis_index("s")
        wid = s * NC + c

        pltpu.async_copy(ei_hbm.at[0, wid, 0], src_sec.at[0], isem_0)
        pltpu.async_copy(ei_hbm.at[1, wid, 0], dst_sec.at[0], isem_0)

        def zfill(i, carry):
            for j in range(D // 16):
                rows_a[i, pl.ds(j * 16, 16)] = jnp.zeros((16,), jnp.float32)
            return carry
        lax.fori_loop(0, CH, zfill, 0)
        for m in range(RPT // CH):
            pltpu.async_copy(rows_a,
                             acc_sh.at[pl.ds(s * RPT + m * CH, CH)], csem)
        if with_cnt:
            def zfill1(i, carry):
                zero_v[pl.ds(i * 16, 16)] = jnp.zeros((16,), jnp.float32)
                return carry
            lax.fori_loop(0, RPT // 16, zfill1, 0)
            pltpu.async_copy(zero_v, cnt_sh.at[pl.ds(s * RPT, RPT)], csem)
            for j in range(CH // 16):
                ones_v[pl.ds(j * 16, 16)] = jnp.ones((16,), jnp.float32)
            pltpu.make_async_copy(zero_v, cnt_sh.at[pl.ds(s * RPT, RPT)],
                                  csem).wait()
        for m in range(RPT // CH):
            pltpu.make_async_copy(rows_a,
                                  acc_sh.at[pl.ds(s * RPT + m * CH, CH)],
                                  csem).wait()
        plsc.subcore_barrier()

        def leg(b, k, rows_v, sem):
            pltpu.make_async_copy(x_hbm.at[src_sec.at[b, 0]], rows_v,
                                  sem).wait()
            pltpu.sync_copy(rows_v, acc_sh.at[dst_sec.at[b, k]], add=True)
            if with_cnt:
                pltpu.async_copy(ones_v, cnt_sh.at[dst_sec.at[b, k]], csem,
                                 add=True)

            @pl.when(k + 2 < SCH)
            def _():
                pltpu.async_copy(x_hbm.at[src_sec.at[b, k + 2]], rows_v, sem)

        for s2 in range(SEC):
            b = s2 % 2
            nb = (s2 + 1) % 2
            if s2 + 1 < SEC:
                pltpu.async_copy(ei_hbm.at[0, wid, s2 + 1], src_sec.at[nb],
                                 isems[nb])
                pltpu.async_copy(ei_hbm.at[1, wid, s2 + 1], dst_sec.at[nb],
                                 isems[nb])
            pltpu.make_async_copy(ei_hbm.at[0, wid, 0], src_sec.at[b],
                                  isems[b]).wait()
            pltpu.make_async_copy(ei_hbm.at[1, wid, 0], dst_sec.at[b],
                                  isems[b]).wait()
            pltpu.async_copy(x_hbm.at[src_sec.at[b, 0]], rows_a, sem_a)
            pltpu.async_copy(x_hbm.at[src_sec.at[b, 1]], rows_b, sem_b)

            def pair(p, carry):
                k0 = 2 * p
                leg(b, k0, rows_a, sem_a)

                @pl.when(k0 + 1 < SCH)
                def _():
                    leg(b, k0 + 1, rows_b, sem_b)
                return carry

            lax.fori_loop(0, (SCH + 1) // 2, pair, 0)
        if with_cnt:
            def drain_cnt(k, carry):
                pltpu.make_async_copy(ones_v, cnt_sh.at[dst_sec.at[0, 0]],
                                      csem).wait()
                return carry
            lax.fori_loop(0, NCH, drain_cnt, 0)
        plsc.subcore_barrier()

        pltpu.sync_copy(acc_sh.at[pl.ds(s * RPT, RPT)],
                        sums_hbm.at[c, pl.ds(s * RPT, RPT)])
        if with_cnt:
            @pl.when(s == 0)
            def _():
                pltpu.sync_copy(cnt_sh, cnt_hbm.at[c])

    return sc


def _make_tc(final: bool):
    R = 2048
    grid = NP // R

    def body(sums_ref, cnt_ref, x_ref, wl_ref, bl_ref, wr_ref, br_ref, o_ref):
        sm = sums_ref[0] + sums_ref[1]
        ct = cnt_ref[0] + cnt_ref[1]
        mean = sm / jnp.clip(ct, 1.0, None)[:, None]
        dn = (((1,), (1,)), ((), ()))
        h = (lax.dot_general(mean, wl_ref[...], dn,
                             precision=lax.Precision.DEFAULT)
             + bl_ref[...]
             + lax.dot_general(x_ref[...], wr_ref[...], dn,
                               precision=lax.Precision.DEFAULT)
             + br_ref[...])
        h = jnp.where(h > 0, h, jnp.exp(jnp.minimum(h, 0.0)) - 1.0)
        if final:
            m = jnp.max(h, axis=-1, keepdims=True)
            sh = h - m
            h = sh - jnp.log(jnp.sum(jnp.exp(sh), axis=-1, keepdims=True))
        o_ref[...] = h

    return pl.pallas_call(
        body,
        grid=(grid,),
        in_specs=[
            pl.BlockSpec((NC, R, D), lambda i: (0, i, 0)),
            pl.BlockSpec((NC, R), lambda i: (0, i)),
            pl.BlockSpec((R, D), lambda i: (i, 0)),
            pl.BlockSpec((D, D), lambda i: (0, 0)),
            pl.BlockSpec((1, D), lambda i: (0, 0)),
            pl.BlockSpec((D, D), lambda i: (0, 0)),
            pl.BlockSpec((1, D), lambda i: (0, 0)),
        ],
        out_specs=pl.BlockSpec((R, D), lambda i: (i, 0)),
        out_shape=jax.ShapeDtypeStruct((N, D), jnp.float32),
    )


def kernel(x, edge_index, W1l, b1l, W1r, b1r, W2l, b2l, W2r, b2r):
    ei = edge_index.reshape(2, NW, SEC, SCH, CH)

    sc1 = _make_sc(with_cnt=True)
    sc2 = _make_sc(with_cnt=False)
    tc1 = _make_tc(final=False)
    tc2 = _make_tc(final=True)

    sums1, cnt = sc1(x, ei)
    h = tc1(sums1, cnt, x, W1l, b1l.reshape(1, D), W1r, b1r.reshape(1, D))
    (sums2,) = sc2(h, ei)
    return tc2(sums2, cnt, h, W2l, b2l.reshape(1, D), W2r, b2r.reshape(1, D))
